# baseline (speedup 1.0000x reference)
import functools

import jax
import jax.numpy as jnp
from jax import lax
from jax.experimental import pallas as pl
from jax.experimental.pallas import tpu as pltpu
from jax.experimental.pallas import tpu_sc as plsc

T = 4096
H = 2048
I = 1408
E = 8
TOPK = 2
BT = 128
BTLOG = 7
NSLOT = TOPK * T + E * BT
NT = NSLOT // BT
NTE = 80
BR = 512
NW = 32
TPW = T // NW
APW = TOPK * TPW


def _take16(vec, idx):
    dn = lax.GatherDimensionNumbers(offset_dims=(), collapsed_slice_dims=(0,),
                                    start_index_map=(0,))
    return lax.gather(vec, idx[:, None], dn, (1,),
                      mode=lax.GatherScatterMode.PROMISE_IN_BOUNDS)


def _cumsum16(x):
    lanes = lax.iota(jnp.int32, 16)
    c = x
    for k in (1, 2, 4, 8):
        sh = _take16(c, jnp.maximum(lanes - k, 0))
        c = c + jnp.where(lanes >= k, sh, 0)
    return c


def _sum16(x):
    return _take16(_cumsum16(x), jnp.full((16,), 15, jnp.int32))


def _router_body(x_ref, gw_ref, eid_ref, wgt_ref):
    xb = x_ref[...]
    gw = gw_ref[...]
    logits = lax.dot_general(xb, gw, (((1,), (1,)), ((), ())),
                             preferred_element_type=jnp.float32)
    m = jnp.max(logits, axis=1, keepdims=True)
    ex = jnp.exp(logits - m)
    p = ex / jnp.sum(ex, axis=1, keepdims=True)
    ii = lax.broadcasted_iota(jnp.int32, p.shape, 1)
    v0 = jnp.max(p, axis=1, keepdims=True)
    i0 = jnp.min(jnp.where(p >= v0, ii, E), axis=1, keepdims=True)
    p2 = jnp.where(ii == i0, -1.0, p)
    v1 = jnp.max(p2, axis=1, keepdims=True)
    i1 = jnp.min(jnp.where(p2 >= v1, ii, E), axis=1, keepdims=True)
    s = v0 + v1
    eid_ref[...] = jnp.concatenate([i0, i1], axis=1)
    wgt_ref[...] = jnp.concatenate([v0 / s, v1 / s], axis=1)


def _router(x, gate_w, interpret=False):
    return pl.pallas_call(
        _router_body,
        grid=(T // BR,),
        in_specs=[
            pl.BlockSpec((BR, H), lambda i: (i, 0)),
            pl.BlockSpec((E, H), lambda i: (0, 0)),
        ],
        out_specs=[
            pl.BlockSpec((BR, TOPK), lambda i: (i, 0)),
            pl.BlockSpec((BR, TOPK), lambda i: (i, 0)),
        ],
        out_shape=[
            jax.ShapeDtypeStruct((T, TOPK), jnp.int32),
            jax.ShapeDtypeStruct((T, TOPK), jnp.float32),
        ],
        interpret=interpret,
    )(x, gate_w)


def _dispatch(eidf):
    mesh = plsc.VectorSubcoreMesh(core_axis_name="c", subcore_axis_name="s")

    @functools.partial(
        pl.kernel, mesh=mesh,
        out_type=[
            jax.ShapeDtypeStruct((TOPK * T,), jnp.int32),
            jax.ShapeDtypeStruct((NTE,), jnp.int32),
        ],
        scratch_types=[
            pltpu.VMEM((TOPK * T,), jnp.int32),
            pltpu.VMEM((TPW,), jnp.int32),
            pltpu.VMEM((TPW,), jnp.int32),
            pltpu.VMEM((NTE,), jnp.int32),
        ],
    )
    def k(eid_hbm, dest_hbm, te_hbm, eid_v, dbuf0, dbuf1, tebuf):
        wid = lax.axis_index("s") * 2 + lax.axis_index("c")
        lanes = lax.iota(jnp.int32, 16)
        pltpu.sync_copy(eid_hbm, eid_v)
        vpw = TPW // 16

        def cnt_body(j, carry):
            tot, pre0, pre1 = carry
            v = eid_v[pl.ds(j * 16, 16)]
            mine0 = lax.broadcast_in_dim(
                lax.convert_element_type(j < wid * vpw, jnp.int32), (16,), ())
            mine1 = lax.broadcast_in_dim(
                lax.convert_element_type(j < (T // 16) + wid * vpw, jnp.int32),
                (16,), ())
            for e in range(E):
                c = _sum16(jnp.where(v == e, 1, 0))
                sel = lanes == e
                tot = tot + jnp.where(sel, c, 0)
                pre0 = pre0 + jnp.where(sel, c * mine0, 0)
                pre1 = pre1 + jnp.where(sel, c * mine1, 0)
            return tot, pre0, pre1

        zero = jnp.zeros((16,), jnp.int32)
        tot, pre0, pre1 = lax.fori_loop(0, (TOPK * T) // 16, cnt_body,
                                        (zero, zero, zero))
        padded = ((tot + (BT - 1)) >> BTLOG) << BTLOG
        ends = _cumsum16(padded)
        off = ends - padded

        @pl.when(wid == 0)
        def _():
            for i3 in range(NTE // 16):
                ii = lanes + i3 * 16
                te = jnp.zeros((16,), jnp.int32)
                for e in range(E):
                    end_b = _take16(ends, jnp.full((16,), e, jnp.int32))
                    te = te + jnp.where(ii * BT >= end_b, 1, 0)
                tebuf[pl.ds(i3 * 16, 16)] = jnp.minimum(
                    te, jnp.full((16,), E - 1, jnp.int32))
            pltpu.sync_copy(tebuf, te_hbm)

        def make_dest(dbuf, src0):
            def dest_body(i, cnt):
                v = eid_v[pl.ds(src0 + i * 16, 16)]
                rank = jnp.zeros((16,), jnp.int32)
                upd = jnp.zeros((16,), jnp.int32)
                for e in range(E):
                    m = v == e
                    mi = jnp.where(m, 1, 0)
                    rank = rank + jnp.where(m, _cumsum16(mi) - 1, 0)
                    upd = upd + jnp.where(lanes == e, _sum16(mi), 0)
                dbuf[pl.ds(i * 16, 16)] = _take16(cnt, v) + rank
                return cnt + upd
            return dest_body

        lax.fori_loop(0, vpw, make_dest(dbuf0, wid * TPW), off + pre0)
        lax.fori_loop(0, vpw, make_dest(dbuf1, T + wid * TPW), off + pre1)
        pltpu.sync_copy(dbuf0, dest_hbm.at[pl.ds(wid * TPW, TPW)])
        pltpu.sync_copy(dbuf1, dest_hbm.at[pl.ds(T + wid * TPW, TPW)])

    return k(eidf)


def _moe_body(te_ref, xs_ref, w1g_ref, w1u_ref, w2_ref, y_ref):
    xb = xs_ref[...]
    g = lax.dot_general(xb, w1g_ref[0], (((1,), (1,)), ((), ())),
                        preferred_element_type=jnp.float32)
    u = lax.dot_general(xb, w1u_ref[0], (((1,), (1,)), ((), ())),
                        preferred_element_type=jnp.float32)
    act = (g * jax.nn.sigmoid(g) * u).astype(jnp.bfloat16)
    y_ref[...] = lax.dot_general(act, w2_ref[0], (((1,), (1,)), ((), ())),
                                 preferred_element_type=jnp.float32)


def _moe_mm(tile_e, xs, w1g, w1u, w2b, interpret=False):
    grid_spec = pltpu.PrefetchScalarGridSpec(
        num_scalar_prefetch=1,
        grid=(NT,),
        in_specs=[
            pl.BlockSpec((BT, H), lambda i, te: (i, 0)),
            pl.BlockSpec((1, I, H), lambda i, te: (te[i], 0, 0)),
            pl.BlockSpec((1, I, H), lambda i, te: (te[i], 1, 0)),
            pl.BlockSpec((1, H, I), lambda i, te: (te[i], 0, 0)),
        ],
        out_specs=pl.BlockSpec((BT, H), lambda i, te: (i, 0)),
    )
    return pl.pallas_call(
        _moe_body,
        grid_spec=grid_spec,
        out_shape=jax.ShapeDtypeStruct((NSLOT, H), jnp.float32),
        interpret=interpret,
    )(tile_e, xs, w1g, w1u, w2b)


def kernel(x, w1, w2, gate_w):
    eid, wgt = _router(x, gate_w)
    destf, te = _dispatch(eid.T.reshape(TOPK * T))
    src = jnp.zeros((NSLOT,), jnp.int32).at[destf].set(
        jnp.arange(TOPK * T, dtype=jnp.int32) % T)
    xs = x.astype(jnp.bfloat16)[src]
    w1b = w1.astype(jnp.bfloat16)
    w2b = w2.astype(jnp.bfloat16)
    y = _moe_mm(te, xs, w1b, w1b, w2b)
    d0 = destf[:T]
    d1 = destf[T:]
    return wgt[:, :1] * y[d0] + wgt[:, 1:] * y[d1]


# speedup vs baseline: 1.2575x; 1.2575x over previous
import functools

import jax
import jax.numpy as jnp
from jax import lax
from jax.experimental import pallas as pl
from jax.experimental.pallas import tpu as pltpu
from jax.experimental.pallas import tpu_sc as plsc

T = 4096
H = 2048
I = 1408
E = 8
TOPK = 2
BT = 512
BTLOG = 9
NSLOT = TOPK * T + E * BT
NT = NSLOT // BT
NTE = 80
BR = 512
NW = 32
TPW = T // NW
APW = TOPK * TPW


def _take16(vec, idx):
    dn = lax.GatherDimensionNumbers(offset_dims=(), collapsed_slice_dims=(0,),
                                    start_index_map=(0,))
    return lax.gather(vec, idx[:, None], dn, (1,),
                      mode=lax.GatherScatterMode.PROMISE_IN_BOUNDS)


def _cumsum16(x):
    lanes = lax.iota(jnp.int32, 16)
    c = x
    for k in (1, 2, 4, 8):
        sh = _take16(c, jnp.maximum(lanes - k, 0))
        c = c + jnp.where(lanes >= k, sh, 0)
    return c


def _sum16(x):
    return _take16(_cumsum16(x), jnp.full((16,), 15, jnp.int32))


def _router_body(x_ref, gw_ref, eid_ref, wgt_ref):
    xb = x_ref[...]
    gw = gw_ref[...]
    logits = lax.dot_general(xb, gw, (((1,), (1,)), ((), ())),
                             preferred_element_type=jnp.float32)
    m = jnp.max(logits, axis=1, keepdims=True)
    ex = jnp.exp(logits - m)
    p = ex / jnp.sum(ex, axis=1, keepdims=True)
    ii = lax.broadcasted_iota(jnp.int32, p.shape, 1)
    v0 = jnp.max(p, axis=1, keepdims=True)
    i0 = jnp.min(jnp.where(p >= v0, ii, E), axis=1, keepdims=True)
    p2 = jnp.where(ii == i0, -1.0, p)
    v1 = jnp.max(p2, axis=1, keepdims=True)
    i1 = jnp.min(jnp.where(p2 >= v1, ii, E), axis=1, keepdims=True)
    s = v0 + v1
    eid_ref[...] = jnp.concatenate([i0, i1], axis=1)
    wgt_ref[...] = jnp.concatenate([v0 / s, v1 / s], axis=1)


def _router(x, gate_w, interpret=False):
    return pl.pallas_call(
        _router_body,
        grid=(T // BR,),
        in_specs=[
            pl.BlockSpec((BR, H), lambda i: (i, 0)),
            pl.BlockSpec((E, H), lambda i: (0, 0)),
        ],
        out_specs=[
            pl.BlockSpec((BR, TOPK), lambda i: (i, 0)),
            pl.BlockSpec((BR, TOPK), lambda i: (i, 0)),
        ],
        out_shape=[
            jax.ShapeDtypeStruct((T, TOPK), jnp.int32),
            jax.ShapeDtypeStruct((T, TOPK), jnp.float32),
        ],
        interpret=interpret,
    )(x, gate_w)


def _dispatch(eidf):
    mesh = plsc.VectorSubcoreMesh(core_axis_name="c", subcore_axis_name="s")

    @functools.partial(
        pl.kernel, mesh=mesh,
        out_type=[
            jax.ShapeDtypeStruct((TOPK * T,), jnp.int32),
            jax.ShapeDtypeStruct((NTE,), jnp.int32),
        ],
        scratch_types=[
            pltpu.VMEM((TOPK * T,), jnp.int32),
            pltpu.VMEM((TPW,), jnp.int32),
            pltpu.VMEM((TPW,), jnp.int32),
            pltpu.VMEM((NTE,), jnp.int32),
        ],
    )
    def k(eid_hbm, dest_hbm, te_hbm, eid_v, dbuf0, dbuf1, tebuf):
        wid = lax.axis_index("s") * 2 + lax.axis_index("c")
        lanes = lax.iota(jnp.int32, 16)
        pltpu.sync_copy(eid_hbm, eid_v)
        vpw = TPW // 16

        def cnt_body(j, carry):
            tot, pre0, pre1 = carry
            v = eid_v[pl.ds(j * 16, 16)]
            mine0 = lax.broadcast_in_dim(
                lax.convert_element_type(j < wid * vpw, jnp.int32), (16,), ())
            mine1 = lax.broadcast_in_dim(
                lax.convert_element_type(j < (T // 16) + wid * vpw, jnp.int32),
                (16,), ())
            for e in range(E):
                c = _sum16(jnp.where(v == e, 1, 0))
                sel = lanes == e
                tot = tot + jnp.where(sel, c, 0)
                pre0 = pre0 + jnp.where(sel, c * mine0, 0)
                pre1 = pre1 + jnp.where(sel, c * mine1, 0)
            return tot, pre0, pre1

        zero = jnp.zeros((16,), jnp.int32)
        tot, pre0, pre1 = lax.fori_loop(0, (TOPK * T) // 16, cnt_body,
                                        (zero, zero, zero))
        padded = ((tot + (BT - 1)) >> BTLOG) << BTLOG
        ends = _cumsum16(padded)
        off = ends - padded

        @pl.when(wid == 0)
        def _():
            for i3 in range(NTE // 16):
                ii = lanes + i3 * 16
                te = jnp.zeros((16,), jnp.int32)
                for e in range(E):
                    end_b = _take16(ends, jnp.full((16,), e, jnp.int32))
                    te = te + jnp.where(ii * BT >= end_b, 1, 0)
                tebuf[pl.ds(i3 * 16, 16)] = jnp.minimum(
                    te, jnp.full((16,), E - 1, jnp.int32))
            pltpu.sync_copy(tebuf, te_hbm)

        def make_dest(dbuf, src0):
            def dest_body(i, cnt):
                v = eid_v[pl.ds(src0 + i * 16, 16)]
                rank = jnp.zeros((16,), jnp.int32)
                upd = jnp.zeros((16,), jnp.int32)
                for e in range(E):
                    m = v == e
                    mi = jnp.where(m, 1, 0)
                    rank = rank + jnp.where(m, _cumsum16(mi) - 1, 0)
                    upd = upd + jnp.where(lanes == e, _sum16(mi), 0)
                dbuf[pl.ds(i * 16, 16)] = _take16(cnt, v) + rank
                return cnt + upd
            return dest_body

        lax.fori_loop(0, vpw, make_dest(dbuf0, wid * TPW), off + pre0)
        lax.fori_loop(0, vpw, make_dest(dbuf1, T + wid * TPW), off + pre1)
        pltpu.sync_copy(dbuf0, dest_hbm.at[pl.ds(wid * TPW, TPW)])
        pltpu.sync_copy(dbuf1, dest_hbm.at[pl.ds(T + wid * TPW, TPW)])

    return k(eidf)


def _moe_body(te_ref, xs_ref, w1g_ref, w1u_ref, w2_ref, y_ref):
    xb = xs_ref[...]
    g = lax.dot_general(xb, w1g_ref[0], (((1,), (1,)), ((), ())),
                        preferred_element_type=jnp.float32)
    u = lax.dot_general(xb, w1u_ref[0], (((1,), (1,)), ((), ())),
                        preferred_element_type=jnp.float32)
    act = (g * jax.nn.sigmoid(g) * u).astype(jnp.bfloat16)
    y_ref[...] = lax.dot_general(act, w2_ref[0], (((1,), (1,)), ((), ())),
                                 preferred_element_type=jnp.float32)


def _moe_mm(tile_e, xs, w1g, w1u, w2b, interpret=False):
    grid_spec = pltpu.PrefetchScalarGridSpec(
        num_scalar_prefetch=1,
        grid=(NT,),
        in_specs=[
            pl.BlockSpec((BT, H), lambda i, te: (i, 0)),
            pl.BlockSpec((1, I, H), lambda i, te: (te[i], 0, 0)),
            pl.BlockSpec((1, I, H), lambda i, te: (te[i], 1, 0)),
            pl.BlockSpec((1, H, I), lambda i, te: (te[i], 0, 0)),
        ],
        out_specs=pl.BlockSpec((BT, H), lambda i, te: (i, 0)),
    )
    return pl.pallas_call(
        _moe_body,
        grid_spec=grid_spec,
        out_shape=jax.ShapeDtypeStruct((NSLOT, H), jnp.float32),
        interpret=interpret,
    )(tile_e, xs, w1g, w1u, w2b)


def kernel(x, w1, w2, gate_w):
    eid, wgt = _router(x, gate_w)
    destf, te = _dispatch(eid.T.reshape(TOPK * T))
    src = jnp.zeros((NSLOT,), jnp.int32).at[destf].set(
        jnp.arange(TOPK * T, dtype=jnp.int32) % T)
    xs = x.astype(jnp.bfloat16)[src]
    w1b = w1.astype(jnp.bfloat16)
    w2b = w2.astype(jnp.bfloat16)
    y = _moe_mm(te, xs, w1b, w1b, w2b)
    d0 = destf[:T]
    d1 = destf[T:]
    return wgt[:, :1] * y[d0] + wgt[:, 1:] * y[d1]


# speedup vs baseline: 1.3273x; 1.0555x over previous
import functools

import jax
import jax.numpy as jnp
from jax import lax
from jax.experimental import pallas as pl
from jax.experimental.pallas import tpu as pltpu
from jax.experimental.pallas import tpu_sc as plsc

T = 4096
H = 2048
I = 1408
E = 8
TOPK = 2
BT = 256
BTLOG = 8
NSLOT = TOPK * T + E * BT
NT = NSLOT // BT
NTE = 80
BR = 512
NW = 32
TPW = T // NW
APW = TOPK * TPW


def _take16(vec, idx):
    dn = lax.GatherDimensionNumbers(offset_dims=(), collapsed_slice_dims=(0,),
                                    start_index_map=(0,))
    return lax.gather(vec, idx[:, None], dn, (1,),
                      mode=lax.GatherScatterMode.PROMISE_IN_BOUNDS)


def _cumsum16(x):
    lanes = lax.iota(jnp.int32, 16)
    c = x
    for k in (1, 2, 4, 8):
        sh = _take16(c, jnp.maximum(lanes - k, 0))
        c = c + jnp.where(lanes >= k, sh, 0)
    return c


def _sum16(x):
    return _take16(_cumsum16(x), jnp.full((16,), 15, jnp.int32))


def _router_body(x_ref, gw_ref, eid_ref, wgt_ref):
    xb = x_ref[...]
    gw = gw_ref[...]
    logits = lax.dot_general(xb, gw, (((1,), (1,)), ((), ())),
                             preferred_element_type=jnp.float32)
    m = jnp.max(logits, axis=1, keepdims=True)
    ex = jnp.exp(logits - m)
    p = ex / jnp.sum(ex, axis=1, keepdims=True)
    ii = lax.broadcasted_iota(jnp.int32, p.shape, 1)
    v0 = jnp.max(p, axis=1, keepdims=True)
    i0 = jnp.min(jnp.where(p >= v0, ii, E), axis=1, keepdims=True)
    p2 = jnp.where(ii == i0, -1.0, p)
    v1 = jnp.max(p2, axis=1, keepdims=True)
    i1 = jnp.min(jnp.where(p2 >= v1, ii, E), axis=1, keepdims=True)
    s = v0 + v1
    eid_ref[...] = jnp.concatenate([i0, i1], axis=1)
    wgt_ref[...] = jnp.concatenate([v0 / s, v1 / s], axis=1)


def _router(x, gate_w, interpret=False):
    return pl.pallas_call(
        _router_body,
        grid=(T // BR,),
        in_specs=[
            pl.BlockSpec((BR, H), lambda i: (i, 0)),
            pl.BlockSpec((E, H), lambda i: (0, 0)),
        ],
        out_specs=[
            pl.BlockSpec((BR, TOPK), lambda i: (i, 0)),
            pl.BlockSpec((BR, TOPK), lambda i: (i, 0)),
        ],
        out_shape=[
            jax.ShapeDtypeStruct((T, TOPK), jnp.int32),
            jax.ShapeDtypeStruct((T, TOPK), jnp.float32),
        ],
        interpret=interpret,
    )(x, gate_w)


def _dispatch(eidf):
    mesh = plsc.VectorSubcoreMesh(core_axis_name="c", subcore_axis_name="s")

    @functools.partial(
        pl.kernel, mesh=mesh,
        out_type=[
            jax.ShapeDtypeStruct((TOPK * T,), jnp.int32),
            jax.ShapeDtypeStruct((NTE,), jnp.int32),
        ],
        scratch_types=[
            pltpu.VMEM((TOPK * T,), jnp.int32),
            pltpu.VMEM((TPW,), jnp.int32),
            pltpu.VMEM((TPW,), jnp.int32),
            pltpu.VMEM((NTE,), jnp.int32),
        ],
    )
    def k(eid_hbm, dest_hbm, te_hbm, eid_v, dbuf0, dbuf1, tebuf):
        wid = lax.axis_index("s") * 2 + lax.axis_index("c")
        lanes = lax.iota(jnp.int32, 16)
        pltpu.sync_copy(eid_hbm, eid_v)
        vpw = TPW // 16

        def cnt_body(j, carry):
            tot, pre0, pre1 = carry
            v = eid_v[pl.ds(j * 16, 16)]
            mine0 = lax.broadcast_in_dim(
                lax.convert_element_type(j < wid * vpw, jnp.int32), (16,), ())
            mine1 = lax.broadcast_in_dim(
                lax.convert_element_type(j < (T // 16) + wid * vpw, jnp.int32),
                (16,), ())
            for e in range(E):
                c = _sum16(jnp.where(v == e, 1, 0))
                sel = lanes == e
                tot = tot + jnp.where(sel, c, 0)
                pre0 = pre0 + jnp.where(sel, c * mine0, 0)
                pre1 = pre1 + jnp.where(sel, c * mine1, 0)
            return tot, pre0, pre1

        zero = jnp.zeros((16,), jnp.int32)
        tot, pre0, pre1 = lax.fori_loop(0, (TOPK * T) // 16, cnt_body,
                                        (zero, zero, zero))
        padded = ((tot + (BT - 1)) >> BTLOG) << BTLOG
        ends = _cumsum16(padded)
        off = ends - padded

        @pl.when(wid == 0)
        def _():
            for i3 in range(NTE // 16):
                ii = lanes + i3 * 16
                te = jnp.zeros((16,), jnp.int32)
                for e in range(E):
                    end_b = _take16(ends, jnp.full((16,), e, jnp.int32))
                    te = te + jnp.where(ii * BT >= end_b, 1, 0)
                tebuf[pl.ds(i3 * 16, 16)] = jnp.minimum(
                    te, jnp.full((16,), E - 1, jnp.int32))
            pltpu.sync_copy(tebuf, te_hbm)

        def make_dest(dbuf, src0):
            def dest_body(i, cnt):
                v = eid_v[pl.ds(src0 + i * 16, 16)]
                rank = jnp.zeros((16,), jnp.int32)
                upd = jnp.zeros((16,), jnp.int32)
                for e in range(E):
                    m = v == e
                    mi = jnp.where(m, 1, 0)
                    rank = rank + jnp.where(m, _cumsum16(mi) - 1, 0)
                    upd = upd + jnp.where(lanes == e, _sum16(mi), 0)
                dbuf[pl.ds(i * 16, 16)] = _take16(cnt, v) + rank
                return cnt + upd
            return dest_body

        lax.fori_loop(0, vpw, make_dest(dbuf0, wid * TPW), off + pre0)
        lax.fori_loop(0, vpw, make_dest(dbuf1, T + wid * TPW), off + pre1)
        pltpu.sync_copy(dbuf0, dest_hbm.at[pl.ds(wid * TPW, TPW)])
        pltpu.sync_copy(dbuf1, dest_hbm.at[pl.ds(T + wid * TPW, TPW)])

    return k(eidf)


def _moe_body(te_ref, xs_ref, w1g_ref, w1u_ref, w2_ref, y_ref):
    xb = xs_ref[...]
    g = lax.dot_general(xb, w1g_ref[0], (((1,), (1,)), ((), ())),
                        preferred_element_type=jnp.float32)
    u = lax.dot_general(xb, w1u_ref[0], (((1,), (1,)), ((), ())),
                        preferred_element_type=jnp.float32)
    act = (g * jax.nn.sigmoid(g) * u).astype(jnp.bfloat16)
    y_ref[...] = lax.dot_general(act, w2_ref[0], (((1,), (1,)), ((), ())),
                                 preferred_element_type=jnp.float32)


def _moe_mm(tile_e, xs, w1g, w1u, w2b, interpret=False):
    grid_spec = pltpu.PrefetchScalarGridSpec(
        num_scalar_prefetch=1,
        grid=(NT,),
        in_specs=[
            pl.BlockSpec((BT, H), lambda i, te: (i, 0)),
            pl.BlockSpec((1, I, H), lambda i, te: (te[i], 0, 0)),
            pl.BlockSpec((1, I, H), lambda i, te: (te[i], 1, 0)),
            pl.BlockSpec((1, H, I), lambda i, te: (te[i], 0, 0)),
        ],
        out_specs=pl.BlockSpec((BT, H), lambda i, te: (i, 0)),
    )
    return pl.pallas_call(
        _moe_body,
        grid_spec=grid_spec,
        out_shape=jax.ShapeDtypeStruct((NSLOT, H), jnp.float32),
        interpret=interpret,
    )(tile_e, xs, w1g, w1u, w2b)


def kernel(x, w1, w2, gate_w):
    eid, wgt = _router(x, gate_w)
    destf, te = _dispatch(eid.T.reshape(TOPK * T))
    src = jnp.zeros((NSLOT,), jnp.int32).at[destf].set(
        jnp.arange(TOPK * T, dtype=jnp.int32) % T)
    xs = x.astype(jnp.bfloat16)[src]
    w1b = w1.astype(jnp.bfloat16)
    w2b = w2.astype(jnp.bfloat16)
    y = _moe_mm(te, xs, w1b, w1b, w2b)
    d0 = destf[:T]
    d1 = destf[T:]
    return wgt[:, :1] * y[d0] + wgt[:, 1:] * y[d1]
